# baseline (speedup 1.0000x reference)
import jax
import jax.numpy as jnp
from jax import lax
from jax.experimental import pallas as pl
from jax.experimental.pallas import tpu as pltpu

FEATURE_DIM = 2048
HIDDEN_DIM = 1024
NUM_EXPERTS = 64
TOP_K = 8
TEMPERATURE = 1.0
BATCH = 8192

BLOCK_M = 2048
SUB_M = 256


def _score_and_select(x, w1, b1, w2, b2, en_n, ts):
    h = lax.dot_general(x, w1, (((1,), (1,)), ((), ())),
                        preferred_element_type=jnp.float32)
    h = jnp.maximum(h + b1, 0.0)
    logits = lax.dot_general(h, w2, (((1,), (1,)), ((), ())),
                             preferred_element_type=jnp.float32)
    logits = (logits + b2) / TEMPERATURE
    m = jnp.max(logits, axis=1, keepdims=True)
    e = jnp.exp(logits - m)
    probs = e / jnp.sum(e, axis=1, keepdims=True)
    xn = x / (jnp.sqrt(jnp.sum(x * x, axis=1, keepdims=True)) + 1e-8)
    raw = lax.dot_general(xn, en_n, (((1,), (1,)), ((), ())),
                          preferred_element_type=jnp.float32)
    sim = (raw + 1.0) * 0.5
    scores = probs * sim * ts
    cols = lax.broadcasted_iota(jnp.int32, scores.shape, 1)
    vals = []
    idxs = []
    for _ in range(TOP_K):
        v = jnp.max(scores, axis=1)
        i = jnp.argmax(scores, axis=1).astype(jnp.int32)
        vals.append(v)
        idxs.append(i)
        scores = jnp.where(cols == i[:, None], -jnp.inf, scores)
    topv = jnp.stack(vals, axis=1)
    topi = jnp.stack(idxs, axis=1)
    topw = topv / (jnp.sum(topv, axis=1, keepdims=True) + 1e-9)
    return topw, topi


def _router_kernel(x_ref, w1_ref, b1_ref, w2_ref, b2_ref, en_ref,
                   trust_ref, stale_ref, w_out_ref, i_out_ref):
    w1 = w1_ref[...]
    b1 = b1_ref[...]
    w2 = w2_ref[...]
    b2 = b2_ref[...]
    en = en_ref[...]
    en_n = en / (jnp.sqrt(jnp.sum(en * en, axis=1, keepdims=True)) + 1e-8)
    ts = trust_ref[...] * stale_ref[...]
    for s in range(BLOCK_M // SUB_M):
        x = x_ref[s * SUB_M:(s + 1) * SUB_M, :]
        topw, topi = _score_and_select(x, w1, b1, w2, b2, en_n, ts)
        w_out_ref[s * SUB_M:(s + 1) * SUB_M, :] = topw
        i_out_ref[s * SUB_M:(s + 1) * SUB_M, :] = topi


@jax.jit
def kernel(x, W1, b1, W2, b2, expert_features, trust, staleness):
    grid = (BATCH // BLOCK_M,)
    fixed = lambda i: (0, 0)
    out = pl.pallas_call(
        _router_kernel,
        grid=grid,
        in_specs=[
            pl.BlockSpec((BLOCK_M, FEATURE_DIM), lambda i: (i, 0)),
            pl.BlockSpec((HIDDEN_DIM, FEATURE_DIM), fixed),
            pl.BlockSpec((1, HIDDEN_DIM), fixed),
            pl.BlockSpec((NUM_EXPERTS, HIDDEN_DIM), fixed),
            pl.BlockSpec((1, NUM_EXPERTS), fixed),
            pl.BlockSpec((NUM_EXPERTS, FEATURE_DIM), fixed),
            pl.BlockSpec((1, NUM_EXPERTS), fixed),
            pl.BlockSpec((1, NUM_EXPERTS), fixed),
        ],
        out_specs=[
            pl.BlockSpec((BLOCK_M, TOP_K), lambda i: (i, 0)),
            pl.BlockSpec((BLOCK_M, TOP_K), lambda i: (i, 0)),
        ],
        out_shape=[
            jax.ShapeDtypeStruct((BATCH, TOP_K), jnp.float32),
            jax.ShapeDtypeStruct((BATCH, TOP_K), jnp.int32),
        ],
        compiler_params=pltpu.CompilerParams(
            dimension_semantics=("parallel",)),
    )(x, W1, b1.reshape(1, -1), W2, b2.reshape(1, -1),
      expert_features, trust.reshape(1, -1), staleness.reshape(1, -1))
    return out[0], out[1]
